# baseline (speedup 1.0000x reference)
import functools

import jax
import jax.numpy as jnp
from jax import lax
from jax.experimental import pallas as pl
from jax.experimental.pallas import tpu as pltpu
from jax.experimental.pallas import tpu_sc as plsc

BINS = 16
LO = 1.0
HI = 80.0
WIDTH = (HI - LO) / BINS
B, N, C = 4, 2048, 3
FEAT = 128
HID = FEAT // 2

S4 = 512
T4 = N - S4
ROWS = 128

NW = 32
RW = B * S4 // NW
WPB = 8
GROUPS = RW // 16

_NF = 256
_FINE = 1.0 / 32.0


def _make_tables():
    import numpy as np
    edges = np.float32(LO) + np.arange(1, BINS, dtype=np.float32) * np.float32(WIDTH)
    e2 = (edges * edges).astype(np.float32)
    tbl = np.zeros(_NF, np.int32)
    e2b = np.full(_NF, np.inf, np.float32)
    for f in range(_NF):
        lo_sq = 32.0 * f
        tbl[f] = np.sum(e2 <= lo_sq)
        inb = [v for v in e2 if lo_sq < v < 32.0 * (f + 1)]
        if inb:
            e2b[f] = inb[0]
    return tbl, e2b


_TBL_NP, _E2B_NP = _make_tables()


def _sc_hist_body(xt_hbm, tbl_hbm, e2b_hbm, counts_hbm,
                  x0_v, x1_v, x2_v, tbl_v, e2b_v, c0_v, c1_v, c2_v, c3_v):
    bufs = (c0_v, c1_v, c2_v, c3_v)
    wid = lax.axis_index("s") * 2 + lax.axis_index("c")
    b = wid // WPB
    i_off = (wid % WPB) * RW

    pltpu.sync_copy(xt_hbm.at[pl.ds((b * C + 0) * N, N)], x0_v)
    pltpu.sync_copy(xt_hbm.at[pl.ds((b * C + 1) * N, N)], x1_v)
    pltpu.sync_copy(xt_hbm.at[pl.ds((b * C + 2) * N, N)], x2_v)
    pltpu.sync_copy(tbl_hbm, tbl_v)
    pltpu.sync_copy(e2b_hbm, e2b_v)

    zeros16 = jnp.zeros((16,), jnp.float32)

    def zrow(r, carry):
        for cb in bufs:
            cb[pl.ds(r * 16, 16)] = zeros16
        return carry

    lax.fori_loop(0, RW * BINS // 16, zrow, 0)

    lane = lax.iota(jnp.int32, 16)
    ones = jnp.ones((16,), jnp.float32)

    def group_body(g, carry):
        base = i_off + g * 16
        xi0 = x0_v[pl.ds(base, 16)]
        xi1 = x1_v[pl.ds(base, 16)]
        xi2 = x2_v[pl.ds(base, 16)]
        rowbase = (g * 16 + lane) * BINS

        def jv_body(jv, c):
            xj0 = x0_v[pl.ds(jv * 16, 16)]
            xj1 = x1_v[pl.ds(jv * 16, 16)]
            xj2 = x2_v[pl.ds(jv * 16, 16)]
            for l in range(16):
                d0 = xi0 - xj0[l]
                d1 = xi1 - xj1[l]
                d2 = xi2 - xj2[l]
                sq = d0 * d0 + d1 * d1 + d2 * d2
                f = jnp.minimum(sq * _FINE, float(_NF - 1)).astype(jnp.int32)
                tb = plsc.load_gather(tbl_v, [f])
                eb = plsc.load_gather(e2b_v, [f])
                idx = tb + (sq >= eb).astype(jnp.int32)
                valid = (sq >= LO * LO) & (sq <= HI * HI)
                plsc.addupdate_scatter(bufs[l % 4], [rowbase + idx], ones,
                                       mask=valid)
            return c

        lax.fori_loop(0, N // 16, jv_body, 0)
        return carry

    lax.fori_loop(0, GROUPS, group_body, 0)

    def mrow(r, carry):
        sl = pl.ds(r * 16, 16)
        c0_v[sl] = (c0_v[sl] + c1_v[sl]) + (c2_v[sl] + c3_v[sl])
        return carry

    lax.fori_loop(0, RW * BINS // 16, mrow, 0)

    pltpu.sync_copy(c0_v, counts_hbm.at[pl.ds(wid * RW * BINS, RW * BINS)])


def _histograms_sc(xt):
    f = functools.partial(
        pl.kernel,
        out_type=jax.ShapeDtypeStruct((B * S4 * BINS,), jnp.float32),
        mesh=plsc.VectorSubcoreMesh(core_axis_name="c", subcore_axis_name="s"),
        compiler_params=pltpu.CompilerParams(needs_layout_passes=False),
        scratch_types=[
            pltpu.VMEM((N,), jnp.float32),
            pltpu.VMEM((N,), jnp.float32),
            pltpu.VMEM((N,), jnp.float32),
            pltpu.VMEM((_NF,), jnp.int32),
            pltpu.VMEM((_NF,), jnp.float32),
            pltpu.VMEM((RW * BINS,), jnp.float32),
            pltpu.VMEM((RW * BINS,), jnp.float32),
            pltpu.VMEM((RW * BINS,), jnp.float32),
            pltpu.VMEM((RW * BINS,), jnp.float32),
        ],
    )(_sc_hist_body)
    return f(xt, jnp.asarray(_TBL_NP), jnp.asarray(_E2B_NP)).reshape(B, S4, BINS)


def _sq_edges():
    import numpy as np
    edges = np.float32(LO) + np.arange(1, BINS, dtype=np.float32) * np.float32(WIDTH)
    return [float(v) for v in (edges * edges).astype(np.float32)]


_SQ_EDGES = _sq_edges()


def _tc_hist_body(xi_ref, xj_ref, counts_ref):
    sq = None
    for c in range(C):
        d = xi_ref[0, :, c:c + 1] - xj_ref[0, c:c + 1, :]
        sq = d * d if sq is None else sq + d * d
    cs = []
    for k in range(BINS + 1):
        if k == 0:
            ge = sq >= (LO * LO)
        elif k == BINS:
            ge = sq > (HI * HI)
        else:
            ge = sq >= _SQ_EDGES[k - 1]
        cs.append(jnp.sum(jnp.where(ge, 1.0, 0.0), axis=1, keepdims=True))
    counts_ref[...] = jnp.concatenate(
        [cs[k] - cs[k + 1] for k in range(BINS)], axis=1)


def _histograms_tc(x, xt2d):
    nb = T4 // ROWS
    return pl.pallas_call(
        _tc_hist_body,
        grid=(B, nb),
        in_specs=[
            pl.BlockSpec((1, ROWS, C), lambda b, r: (b, (S4 // ROWS) + r, 0)),
            pl.BlockSpec((1, C, N), lambda b, r: (b, 0, 0)),
        ],
        out_specs=pl.BlockSpec((ROWS, BINS), lambda b, r: (b * nb + r, 0)),
        out_shape=jax.ShapeDtypeStruct((B * T4, BINS), jnp.float32),
    )(x, xt2d).reshape(B, T4, BINS)


def _mlp_body(csc_ref, ctc_ref, W1_ref, b1_ref, g1_ref, be1_ref,
              W2_ref, b2_ref, g2_ref, be2_ref, h2_ref):
    parts = []
    for b in range(B):
        parts.append(csc_ref[b])
        parts.append(ctc_ref[b])
    counts = jnp.concatenate(parts, axis=0)
    s = jnp.sum(counts, axis=1, keepdims=True)
    hist = counts / s

    def bn(z, g, be):
        m = jnp.mean(z, axis=1, keepdims=True)
        v = jnp.mean((z - m) * (z - m), axis=1, keepdims=True)
        return (z - m) / jnp.sqrt(v + 1e-5) * g + be

    z1 = jax.lax.dot_general(W1_ref[...], hist, (((1,), (1,)), ((), ())),
                             preferred_element_type=jnp.float32) + b1_ref[...]
    h1 = jax.nn.relu(bn(z1, g1_ref[...], be1_ref[...]))
    z2 = jax.lax.dot_general(W2_ref[...], h1, (((1,), (0,)), ((), ())),
                             preferred_element_type=jnp.float32) + b2_ref[...]
    h2_ref[...] = jax.nn.relu(bn(z2, g2_ref[...], be2_ref[...]))


def _out_body(h2_ref, fea_ref, W3_ref, b3_ref, out_ref):
    z3 = jax.lax.dot_general(W3_ref[...], h2_ref[...], (((1,), (0,)), ((), ())),
                             preferred_element_type=jnp.float32) + b3_ref[...]
    out_ref[0] = fea_ref[0] + z3


def kernel(x, fea, W1, b1, g1, be1, W2, b2, g2, be2, W3, b3):
    xt = jnp.transpose(x, (0, 2, 1))
    counts_sc = _histograms_sc(xt.reshape(-1))
    counts_tc = _histograms_tc(x, xt)
    h2 = pl.pallas_call(
        _mlp_body,
        out_shape=jax.ShapeDtypeStruct((HID, B * N), jnp.float32),
    )(counts_sc, counts_tc, W1, b1.reshape(HID, 1), g1.reshape(HID, 1),
      be1.reshape(HID, 1), W2, b2.reshape(HID, 1), g2.reshape(HID, 1),
      be2.reshape(HID, 1))
    out = pl.pallas_call(
        _out_body,
        grid=(B,),
        in_specs=[
            pl.BlockSpec((HID, N), lambda b: (0, b)),
            pl.BlockSpec((1, FEAT, N), lambda b: (b, 0, 0)),
            pl.BlockSpec((FEAT, HID), lambda b: (0, 0)),
            pl.BlockSpec((FEAT, 1), lambda b: (0, 0)),
        ],
        out_specs=pl.BlockSpec((1, FEAT, N), lambda b: (b, 0, 0)),
        out_shape=jax.ShapeDtypeStruct((B, FEAT, N), jnp.float32),
    )(h2, fea, W3, b3.reshape(FEAT, 1))
    return out


# speedup vs baseline: 1.0234x; 1.0234x over previous
import functools

import jax
import jax.numpy as jnp
from jax import lax
from jax.experimental import pallas as pl
from jax.experimental.pallas import tpu as pltpu
from jax.experimental.pallas import tpu_sc as plsc

BINS = 16
LO = 1.0
HI = 80.0
WIDTH = (HI - LO) / BINS
B, N, C = 4, 2048, 3
FEAT = 128
HID = FEAT // 2

S4 = 512
T4 = N - S4
ROWS = 128

NW = 32
RW = B * S4 // NW
WPB = 8
GROUPS = RW // 16

_NF = 256
_FINE = 1.0 / 32.0


def _make_tables():
    import numpy as np
    edges = np.float32(LO) + np.arange(1, BINS, dtype=np.float32) * np.float32(WIDTH)
    e2 = (edges * edges).astype(np.float32)
    tbl = np.zeros(_NF, np.int32)
    e2b = np.full(_NF, np.inf, np.float32)
    for f in range(_NF):
        lo_sq = 32.0 * f
        tbl[f] = np.sum(e2 <= lo_sq)
        inb = [v for v in e2 if lo_sq < v < 32.0 * (f + 1)]
        if inb:
            e2b[f] = inb[0]
    return tbl, e2b


_TBL_NP, _E2B_NP = _make_tables()


def _sc_hist_body(xt_hbm, tbl_hbm, e2b_hbm, counts_hbm,
                  x0_v, x1_v, x2_v, tbl_v, e2b_v, c0_v, c1_v, c2_v, c3_v):
    bufs = (c0_v, c1_v, c2_v, c3_v)
    wid = lax.axis_index("s") * 2 + lax.axis_index("c")
    b = wid // WPB
    i_off = (wid % WPB) * RW

    pltpu.sync_copy(xt_hbm.at[pl.ds((b * C + 0) * N, N)], x0_v)
    pltpu.sync_copy(xt_hbm.at[pl.ds((b * C + 1) * N, N)], x1_v)
    pltpu.sync_copy(xt_hbm.at[pl.ds((b * C + 2) * N, N)], x2_v)
    pltpu.sync_copy(tbl_hbm, tbl_v)
    pltpu.sync_copy(e2b_hbm, e2b_v)

    zeros16 = jnp.zeros((16,), jnp.float32)

    def zrow(r, carry):
        for cb in bufs:
            cb[pl.ds(r * 16, 16)] = zeros16
        return carry

    lax.fori_loop(0, RW * BINS // 16, zrow, 0)

    lane = lax.iota(jnp.int32, 16)
    ones = jnp.ones((16,), jnp.float32)

    def group_body(g, carry):
        base = i_off + g * 16
        xi0 = x0_v[pl.ds(base, 16)]
        xi1 = x1_v[pl.ds(base, 16)]
        xi2 = x2_v[pl.ds(base, 16)]
        rowbase = (g * 16 + lane) * BINS

        @functools.partial(plsc.parallel_loop, 0, N // 16, unroll=2)
        def jv_body(jv):
            xj0 = x0_v[pl.ds(jv * 16, 16)]
            xj1 = x1_v[pl.ds(jv * 16, 16)]
            xj2 = x2_v[pl.ds(jv * 16, 16)]
            for l in range(16):
                d0 = xi0 - xj0[l]
                d1 = xi1 - xj1[l]
                d2 = xi2 - xj2[l]
                sq = d0 * d0 + d1 * d1 + d2 * d2
                f = jnp.minimum(sq * _FINE, float(_NF - 1)).astype(jnp.int32)
                tb = plsc.load_gather(tbl_v, [f])
                eb = plsc.load_gather(e2b_v, [f])
                idx = tb + (sq >= eb).astype(jnp.int32)
                valid = (sq >= LO * LO) & (sq <= HI * HI)
                plsc.addupdate_scatter(bufs[l % 4], [rowbase + idx], ones,
                                       mask=valid)

        return carry

    lax.fori_loop(0, GROUPS, group_body, 0)

    def mrow(r, carry):
        sl = pl.ds(r * 16, 16)
        c0_v[sl] = (c0_v[sl] + c1_v[sl]) + (c2_v[sl] + c3_v[sl])
        return carry

    lax.fori_loop(0, RW * BINS // 16, mrow, 0)

    pltpu.sync_copy(c0_v, counts_hbm.at[pl.ds(wid * RW * BINS, RW * BINS)])


def _histograms_sc(xt):
    f = functools.partial(
        pl.kernel,
        out_type=jax.ShapeDtypeStruct((B * S4 * BINS,), jnp.float32),
        mesh=plsc.VectorSubcoreMesh(core_axis_name="c", subcore_axis_name="s"),
        compiler_params=pltpu.CompilerParams(needs_layout_passes=False),
        scratch_types=[
            pltpu.VMEM((N,), jnp.float32),
            pltpu.VMEM((N,), jnp.float32),
            pltpu.VMEM((N,), jnp.float32),
            pltpu.VMEM((_NF,), jnp.int32),
            pltpu.VMEM((_NF,), jnp.float32),
            pltpu.VMEM((RW * BINS,), jnp.float32),
            pltpu.VMEM((RW * BINS,), jnp.float32),
            pltpu.VMEM((RW * BINS,), jnp.float32),
            pltpu.VMEM((RW * BINS,), jnp.float32),
        ],
    )(_sc_hist_body)
    return f(xt, jnp.asarray(_TBL_NP), jnp.asarray(_E2B_NP)).reshape(B, S4, BINS)


def _sq_edges():
    import numpy as np
    edges = np.float32(LO) + np.arange(1, BINS, dtype=np.float32) * np.float32(WIDTH)
    return [float(v) for v in (edges * edges).astype(np.float32)]


_SQ_EDGES = _sq_edges()


def _tc_hist_body(xi_ref, xj_ref, counts_ref):
    sq = None
    for c in range(C):
        d = xi_ref[0, :, c:c + 1] - xj_ref[0, c:c + 1, :]
        sq = d * d if sq is None else sq + d * d
    cs = []
    for k in range(BINS + 1):
        if k == 0:
            ge = sq >= (LO * LO)
        elif k == BINS:
            ge = sq > (HI * HI)
        else:
            ge = sq >= _SQ_EDGES[k - 1]
        cs.append(jnp.sum(jnp.where(ge, 1.0, 0.0), axis=1, keepdims=True))
    counts_ref[...] = jnp.concatenate(
        [cs[k] - cs[k + 1] for k in range(BINS)], axis=1)


def _histograms_tc(x, xt2d):
    nb = T4 // ROWS
    return pl.pallas_call(
        _tc_hist_body,
        grid=(B, nb),
        in_specs=[
            pl.BlockSpec((1, ROWS, C), lambda b, r: (b, (S4 // ROWS) + r, 0)),
            pl.BlockSpec((1, C, N), lambda b, r: (b, 0, 0)),
        ],
        out_specs=pl.BlockSpec((ROWS, BINS), lambda b, r: (b * nb + r, 0)),
        out_shape=jax.ShapeDtypeStruct((B * T4, BINS), jnp.float32),
    )(x, xt2d).reshape(B, T4, BINS)


def _mlp_body(csc_ref, ctc_ref, fea_ref, W1_ref, b1_ref, g1_ref, be1_ref,
              W2_ref, b2_ref, g2_ref, be2_ref, W3_ref, b3_ref, out_ref):
    parts = []
    for b in range(B):
        parts.append(csc_ref[b])
        parts.append(ctc_ref[b])
    counts = jnp.concatenate(parts, axis=0)
    s = jnp.sum(counts, axis=1, keepdims=True)
    hist = counts / s

    def bn(z, g, be):
        m = jnp.mean(z, axis=1, keepdims=True)
        v = jnp.mean((z - m) * (z - m), axis=1, keepdims=True)
        return (z - m) / jnp.sqrt(v + 1e-5) * g + be

    z1 = jax.lax.dot_general(W1_ref[...], hist, (((1,), (1,)), ((), ())),
                             preferred_element_type=jnp.float32) + b1_ref[...]
    h1 = jax.nn.relu(bn(z1, g1_ref[...], be1_ref[...]))
    z2 = jax.lax.dot_general(W2_ref[...], h1, (((1,), (0,)), ((), ())),
                             preferred_element_type=jnp.float32) + b2_ref[...]
    h2 = jax.nn.relu(bn(z2, g2_ref[...], be2_ref[...]))
    z3 = jax.lax.dot_general(W3_ref[...], h2, (((1,), (0,)), ((), ())),
                             preferred_element_type=jnp.float32) + b3_ref[...]
    for b in range(B):
        out_ref[b] = fea_ref[b] + z3[:, b * N:(b + 1) * N]


def kernel(x, fea, W1, b1, g1, be1, W2, b2, g2, be2, W3, b3):
    xt = jnp.transpose(x, (0, 2, 1))
    counts_sc = _histograms_sc(xt.reshape(-1))
    counts_tc = _histograms_tc(x, xt)
    out = pl.pallas_call(
        _mlp_body,
        out_shape=jax.ShapeDtypeStruct((B, FEAT, N), jnp.float32),
    )(counts_sc, counts_tc, fea, W1, b1.reshape(HID, 1), g1.reshape(HID, 1),
      be1.reshape(HID, 1), W2, b2.reshape(HID, 1), g2.reshape(HID, 1),
      be2.reshape(HID, 1), W3, b3.reshape(FEAT, 1))
    return out


# speedup vs baseline: 2.4876x; 2.4307x over previous
import functools

import jax
import jax.numpy as jnp
from jax import lax
from jax.experimental import pallas as pl
from jax.experimental.pallas import tpu as pltpu
from jax.experimental.pallas import tpu_sc as plsc

BINS = 16
LO = 1.0
HI = 80.0
WIDTH = (HI - LO) / BINS
B, N, C = 4, 2048, 3
FEAT = 128
HID = FEAT // 2

S4 = 1792
T4 = N - S4
ROWS = 128

NW = 32
RW = B * S4 // NW
WPB = 8
GROUPS = RW // 16

_NF = 256
_FINE = 1.0 / 32.0


def _make_tables():
    import numpy as np
    edges = np.float32(LO) + np.arange(1, BINS, dtype=np.float32) * np.float32(WIDTH)
    e2 = (edges * edges).astype(np.float32)
    tbl = np.zeros(_NF, np.int32)
    e2b = np.full(_NF, np.inf, np.float32)
    for f in range(_NF):
        lo_sq = 32.0 * f
        tbl[f] = np.sum(e2 <= lo_sq)
        inb = [v for v in e2 if lo_sq < v < 32.0 * (f + 1)]
        if inb:
            e2b[f] = inb[0]
    return tbl, e2b


_TBL_NP, _E2B_NP = _make_tables()


def _sc_hist_body(xt_hbm, tbl_hbm, e2b_hbm, counts_hbm,
                  x0_v, x1_v, x2_v, tbl_v, e2b_v, c0_v, c1_v, c2_v, c3_v):
    bufs = (c0_v, c1_v, c2_v, c3_v)
    wid = lax.axis_index("s") * 2 + lax.axis_index("c")
    b = wid // WPB
    i_off = (wid % WPB) * RW

    pltpu.sync_copy(xt_hbm.at[pl.ds((b * C + 0) * N, N)], x0_v)
    pltpu.sync_copy(xt_hbm.at[pl.ds((b * C + 1) * N, N)], x1_v)
    pltpu.sync_copy(xt_hbm.at[pl.ds((b * C + 2) * N, N)], x2_v)
    pltpu.sync_copy(tbl_hbm, tbl_v)
    pltpu.sync_copy(e2b_hbm, e2b_v)

    zeros16 = jnp.zeros((16,), jnp.float32)

    def zrow(r, carry):
        for cb in bufs:
            cb[pl.ds(r * 16, 16)] = zeros16
        return carry

    lax.fori_loop(0, RW * BINS // 16, zrow, 0)

    lane = lax.iota(jnp.int32, 16)
    ones = jnp.ones((16,), jnp.float32)

    def group_body(g, carry):
        base = i_off + g * 16
        xi0 = x0_v[pl.ds(base, 16)]
        xi1 = x1_v[pl.ds(base, 16)]
        xi2 = x2_v[pl.ds(base, 16)]
        rowbase = (g * 16 + lane) * BINS

        @functools.partial(plsc.parallel_loop, 0, N // 16, unroll=2)
        def jv_body(jv):
            xj0 = x0_v[pl.ds(jv * 16, 16)]
            xj1 = x1_v[pl.ds(jv * 16, 16)]
            xj2 = x2_v[pl.ds(jv * 16, 16)]
            for l in range(16):
                d0 = xi0 - xj0[l]
                d1 = xi1 - xj1[l]
                d2 = xi2 - xj2[l]
                sq = d0 * d0 + d1 * d1 + d2 * d2
                f = jnp.minimum(sq * _FINE, float(_NF - 1)).astype(jnp.int32)
                tb = plsc.load_gather(tbl_v, [f])
                eb = plsc.load_gather(e2b_v, [f])
                idx = tb + (sq >= eb).astype(jnp.int32)
                valid = (sq >= LO * LO) & (sq <= HI * HI)
                plsc.addupdate_scatter(bufs[l % 4], [rowbase + idx], ones,
                                       mask=valid)

        return carry

    lax.fori_loop(0, GROUPS, group_body, 0)

    def mrow(r, carry):
        sl = pl.ds(r * 16, 16)
        c0_v[sl] = (c0_v[sl] + c1_v[sl]) + (c2_v[sl] + c3_v[sl])
        return carry

    lax.fori_loop(0, RW * BINS // 16, mrow, 0)

    pltpu.sync_copy(c0_v, counts_hbm.at[pl.ds(wid * RW * BINS, RW * BINS)])


def _histograms_sc(xt):
    f = functools.partial(
        pl.kernel,
        out_type=jax.ShapeDtypeStruct((B * S4 * BINS,), jnp.float32),
        mesh=plsc.VectorSubcoreMesh(core_axis_name="c", subcore_axis_name="s"),
        compiler_params=pltpu.CompilerParams(needs_layout_passes=False),
        scratch_types=[
            pltpu.VMEM((N,), jnp.float32),
            pltpu.VMEM((N,), jnp.float32),
            pltpu.VMEM((N,), jnp.float32),
            pltpu.VMEM((_NF,), jnp.int32),
            pltpu.VMEM((_NF,), jnp.float32),
            pltpu.VMEM((RW * BINS,), jnp.float32),
            pltpu.VMEM((RW * BINS,), jnp.float32),
            pltpu.VMEM((RW * BINS,), jnp.float32),
            pltpu.VMEM((RW * BINS,), jnp.float32),
        ],
    )(_sc_hist_body)
    return f(xt, jnp.asarray(_TBL_NP), jnp.asarray(_E2B_NP)).reshape(B, S4, BINS)


def _sq_edges():
    import numpy as np
    edges = np.float32(LO) + np.arange(1, BINS, dtype=np.float32) * np.float32(WIDTH)
    return [float(v) for v in (edges * edges).astype(np.float32)]


_SQ_EDGES = _sq_edges()


def _tc_hist_body(xi_ref, xj_ref, counts_ref):
    sq = None
    for c in range(C):
        d = xi_ref[0, :, c:c + 1] - xj_ref[0, c:c + 1, :]
        sq = d * d if sq is None else sq + d * d
    cs = []
    for k in range(BINS + 1):
        if k == 0:
            ge = sq >= (LO * LO)
        elif k == BINS:
            ge = sq > (HI * HI)
        else:
            ge = sq >= _SQ_EDGES[k - 1]
        cs.append(jnp.sum(jnp.where(ge, 1.0, 0.0), axis=1, keepdims=True))
    counts_ref[...] = jnp.concatenate(
        [cs[k] - cs[k + 1] for k in range(BINS)], axis=1)


def _histograms_tc(x, xt2d):
    nb = T4 // ROWS
    return pl.pallas_call(
        _tc_hist_body,
        grid=(B, nb),
        in_specs=[
            pl.BlockSpec((1, ROWS, C), lambda b, r: (b, (S4 // ROWS) + r, 0)),
            pl.BlockSpec((1, C, N), lambda b, r: (b, 0, 0)),
        ],
        out_specs=pl.BlockSpec((ROWS, BINS), lambda b, r: (b * nb + r, 0)),
        out_shape=jax.ShapeDtypeStruct((B * T4, BINS), jnp.float32),
    )(x, xt2d).reshape(B, T4, BINS)


def _mlp_body(csc_ref, ctc_ref, fea_ref, W1_ref, b1_ref, g1_ref, be1_ref,
              W2_ref, b2_ref, g2_ref, be2_ref, W3_ref, b3_ref, out_ref):
    parts = []
    for b in range(B):
        parts.append(csc_ref[b])
        parts.append(ctc_ref[b])
    counts = jnp.concatenate(parts, axis=0)
    s = jnp.sum(counts, axis=1, keepdims=True)
    hist = counts / s

    def bn(z, g, be):
        m = jnp.mean(z, axis=1, keepdims=True)
        v = jnp.mean((z - m) * (z - m), axis=1, keepdims=True)
        return (z - m) / jnp.sqrt(v + 1e-5) * g + be

    z1 = jax.lax.dot_general(W1_ref[...], hist, (((1,), (1,)), ((), ())),
                             preferred_element_type=jnp.float32) + b1_ref[...]
    h1 = jax.nn.relu(bn(z1, g1_ref[...], be1_ref[...]))
    z2 = jax.lax.dot_general(W2_ref[...], h1, (((1,), (0,)), ((), ())),
                             preferred_element_type=jnp.float32) + b2_ref[...]
    h2 = jax.nn.relu(bn(z2, g2_ref[...], be2_ref[...]))
    z3 = jax.lax.dot_general(W3_ref[...], h2, (((1,), (0,)), ((), ())),
                             preferred_element_type=jnp.float32) + b3_ref[...]
    for b in range(B):
        out_ref[b] = fea_ref[b] + z3[:, b * N:(b + 1) * N]


def kernel(x, fea, W1, b1, g1, be1, W2, b2, g2, be2, W3, b3):
    xt = jnp.transpose(x, (0, 2, 1))
    counts_sc = _histograms_sc(xt.reshape(-1))
    counts_tc = _histograms_tc(x, xt)
    out = pl.pallas_call(
        _mlp_body,
        out_shape=jax.ShapeDtypeStruct((B, FEAT, N), jnp.float32),
    )(counts_sc, counts_tc, fea, W1, b1.reshape(HID, 1), g1.reshape(HID, 1),
      be1.reshape(HID, 1), W2, b2.reshape(HID, 1), g2.reshape(HID, 1),
      be2.reshape(HID, 1), W3, b3.reshape(FEAT, 1))
    return out
